# baseline (speedup 1.0000x reference)
import functools
import math

import jax
import jax.numpy as jnp
from jax import lax
from jax.experimental import pallas as pl
from jax.experimental.pallas import tpu as pltpu
from jax.experimental.pallas import tpu_sc as plsc

N_NODES = 10000
MUL0 = 16
MUL1 = 4
D_IN = MUL0 + 3 * MUL1
W_N = 400

_INV_SQRT3 = 1.0 / math.sqrt(3.0)
_INV_SQRT20 = 1.0 / math.sqrt(float(MUL0 + MUL1))

_NW = 32
_CHUNK = 256


def _edge_body(n_edges, blk, ea_ref, sh_ref, xg_ref, w1_ref, b1_ref,
               w2t_ref, o_ref):
    R = blk // 4
    ea = ea_ref[...]
    h = jnp.maximum(
        jnp.dot(ea, w1_ref[...], preferred_element_type=jnp.float32)
        + b1_ref[...], 0.0)
    ones = jnp.ones((h.shape[0], 1), jnp.float32)
    ht = jnp.concatenate([h, ones], axis=1).T
    wt = jnp.dot(w2t_ref[...], ht, preferred_element_type=jnp.float32)

    xp = xg_ref[...].T
    xgt = jnp.concatenate([xp[32 * k:32 * k + 32] for k in range(4)], axis=1)
    sht = sh_ref[...].T
    ss = sht[0:1]
    xs = xgt[0:16]
    a_s = xs * ss
    dots = jnp.concatenate(
        [xgt[16 + 3 * i:17 + 3 * i] * sht[1:2]
         + xgt[17 + 3 * i:18 + 3 * i] * sht[2:3]
         + xgt[18 + 3 * i:19 + 3 * i] * sht[3:4] for i in range(4)],
        axis=0) * _INV_SQRT3

    out0 = a_s[0:1] * wt[0:16]
    for i in range(1, 16):
        out0 = out0 + a_s[i:i + 1] * wt[16 * i:16 * i + 16]
    for i in range(4):
        out0 = out0 + dots[i:i + 1] * wt[256 + 16 * i:256 + 16 * i + 16]
    out0 = out0 * _INV_SQRT20

    t3 = xs[0:1] * wt[320:324]
    for i in range(1, 16):
        t3 = t3 + xs[i:i + 1] * wt[320 + 4 * i:320 + 4 * i + 4]
    t4 = []
    for m in range(3):
        acc = xgt[16 + m:17 + m] * wt[384:388]
        for i in range(1, 4):
            acc = acc + xgt[16 + 3 * i + m:17 + 3 * i + m] \
                * wt[384 + 4 * i:384 + 4 * i + 4]
        t4.append(acc)

    base = pl.program_id(0) * blk
    eid = base + lax.broadcasted_iota(jnp.int32, (1, blk), 1)
    mask = jnp.where(eid < n_edges, 1.0, 0.0).astype(jnp.float32)
    rows = [out0]
    for j in range(4):
        for m in range(3):
            rows.append((t3[j:j + 1] * sht[1 + m:2 + m]
                         + t4[m][j:j + 1] * ss) * _INV_SQRT20)
    rows.append(mask)
    rows.append(jnp.zeros((3, blk), jnp.float32))
    ot = jnp.concatenate(rows, axis=0) * mask
    op = jnp.concatenate([ot[:, k * R:(k + 1) * R] for k in range(4)], axis=0)
    o_ref[...] = op.T


def _edge_call(ea, sh, xg_p, W1, b1, W2Ta, n_edges, blk, e_pad, interpret=False):
    grid = e_pad // blk
    n_valid = n_edges // blk
    clamp = lambda i: (jnp.minimum(i, n_valid - 1), 0)
    return pl.pallas_call(
        functools.partial(_edge_body, n_edges, blk),
        grid=(grid,),
        in_specs=[
            pl.BlockSpec((blk, 32), clamp),
            pl.BlockSpec((blk, 4), clamp),
            pl.BlockSpec((blk // 4, 128), lambda i: (i, 0)),
            pl.BlockSpec((32, 32), lambda i: (0, 0)),
            pl.BlockSpec((1, 32), lambda i: (0, 0)),
            pl.BlockSpec((400, 33), lambda i: (0, 0)),
        ],
        out_specs=pl.BlockSpec((blk // 4, 128), lambda i: (i, 0)),
        out_shape=jax.ShapeDtypeStruct((e_pad // 4, 128), jnp.float32),
        interpret=interpret,
    )(ea, sh, xg_p, W1, b1, W2Ta)


def _final_body(acc_ref, na_ref, bnw_ref, bnb_ref, sel_ref, selt_ref, o_ref):
    acc = acc_ref[0] + acc_ref[1]
    cnt = jnp.maximum(acc[:, 28:29], 1.0)
    msg = acc[:, :28] / cnt
    out = msg + na_ref[...]
    s = out[:, :16]
    n = out.shape[0]
    mean = jnp.mean(s, axis=0, keepdims=True)
    var = jnp.mean(s * s, axis=0, keepdims=True) - mean * mean
    sn = (s - mean) * jax.lax.rsqrt(var + 1e-5) * bnw_ref[...][:, :16] \
        + bnb_ref[...]
    v = out[:, 16:28]
    fn = jnp.dot(jnp.sum(v * v, axis=0, keepdims=True), sel_ref[...],
                 preferred_element_type=jnp.float32) / (3.0 * n)
    scale4 = bnw_ref[...][:, 16:20] * jax.lax.rsqrt(fn + 1e-5)
    scale12 = jnp.dot(scale4, selt_ref[...],
                      preferred_element_type=jnp.float32)
    o_ref[...] = jnp.concatenate([sn, v * scale12], axis=1)


def _final_call(acc2, node_attr, bnw, bnb, interpret=False):
    sel = jnp.repeat(jnp.eye(4, dtype=jnp.float32), 3, axis=0)
    return pl.pallas_call(
        _final_body,
        out_shape=jax.ShapeDtypeStruct((N_NODES, 28), jnp.float32),
        interpret=interpret,
    )(acc2, node_attr, bnw, bnb, sel, sel.T)


_NBUF = 4


def _gather_call(nap_pad, dst3, e_pad, interpret=False):
    n_chunks = e_pad // (_NW * _CHUNK)
    n_grp = n_chunks // _NBUF
    mesh = plsc.VectorSubcoreMesh(core_axis_name="c", subcore_axis_name="s")

    @functools.partial(
        pl.kernel, mesh=mesh,
        out_type=jax.ShapeDtypeStruct((e_pad, 32), jnp.float32),
        scratch_types=[
            pltpu.VMEM((n_chunks, _CHUNK), jnp.int32),
            pltpu.VMEM((_NBUF, _CHUNK, 32), jnp.float32),
            pltpu.SemaphoreType.DMA((_NBUF,)),
            pltpu.SemaphoreType.DMA((_NBUF,)),
        ],
        compiler_params=pltpu.CompilerParams(use_tc_tiling_on_sc=False),
        interpret=interpret,
    )
    def k(nap_hbm, dst_hbm, out_hbm, idx_v, rows_v, gsem, osem):
        c = lax.axis_index("c")
        s = lax.axis_index("s")
        w = c * 16 + s
        pltpu.sync_copy(dst_hbm.at[w], idx_v)
        base = w * (n_chunks * _CHUNK)

        def fire(j, b):
            pltpu.async_copy(nap_hbm.at[idx_v.at[j]], rows_v.at[b],
                             gsem.at[b])

        for b in range(_NBUF):
            fire(b, b)

        def body(g, _):
            for b in range(_NBUF):
                j = g * _NBUF + b
                pltpu.make_async_copy(nap_hbm.at[idx_v.at[j]], rows_v.at[b],
                                      gsem.at[b]).wait()
                dst = out_hbm.at[pl.ds(base + j * _CHUNK, _CHUNK)]
                pltpu.async_copy(rows_v.at[b], dst, osem.at[b])

                @pl.when(j + _NBUF < n_chunks)
                def _():
                    pltpu.make_async_copy(rows_v.at[b], dst, osem.at[b]).wait()
                    fire(j + _NBUF, b)
            return 0

        lax.fori_loop(0, n_grp, body, 0)
        for b in range(_NBUF):
            j = (n_grp - 1) * _NBUF + b
            dst = out_hbm.at[pl.ds(base + j * _CHUNK, _CHUNK)]
            pltpu.make_async_copy(rows_v.at[b], dst, osem.at[b]).wait()

    return k(nap_pad, dst3)


def _scatter_call(tp, src3, zeros_n, e_pad, interpret=False):
    n_chunks = e_pad // (_NW * _CHUNK)
    rows_per_tile = N_NODES // 16
    mesh = plsc.VectorSubcoreMesh(core_axis_name="c", subcore_axis_name="s")

    @functools.partial(
        pl.kernel, mesh=mesh,
        out_type=jax.ShapeDtypeStruct((2, N_NODES, 32), jnp.float32),
        scratch_types=[
            pltpu.VMEM((n_chunks, _CHUNK), jnp.int32),
            pltpu.VMEM((_NBUF, _CHUNK, 32), jnp.float32),
            pltpu.VMEM_SHARED((N_NODES, 32), jnp.float32),
            pltpu.SemaphoreType.DMA((_NBUF,)),
            pltpu.SemaphoreType.DMA((_NBUF,)),
        ],
        compiler_params=pltpu.CompilerParams(use_tc_tiling_on_sc=False),
        interpret=interpret,
    )
    def k(tp_hbm, src_hbm, zeros_hbm, out_hbm, idx_v, rows_v, acc, gsem, ssem):
        c = lax.axis_index("c")
        s = lax.axis_index("s")
        w = c * 16 + s
        pltpu.sync_copy(zeros_hbm.at[pl.ds(s * rows_per_tile, rows_per_tile)],
                        acc.at[pl.ds(s * rows_per_tile, rows_per_tile)])
        plsc.subcore_barrier()
        pltpu.sync_copy(src_hbm.at[w], idx_v)
        base = w * (n_chunks * _CHUNK)
        n_grp = n_chunks // _NBUF

        def fire(j, b):
            pltpu.async_copy(tp_hbm.at[pl.ds(base + j * _CHUNK, _CHUNK)],
                             rows_v.at[b], gsem.at[b])

        for b in range(_NBUF):
            fire(b, b)

        def body(g, _):
            for b in range(_NBUF):
                j = g * _NBUF + b
                pltpu.make_async_copy(
                    tp_hbm.at[pl.ds(base + j * _CHUNK, _CHUNK)],
                    rows_v.at[b], gsem.at[b]).wait()
                dst = acc.at[idx_v.at[j]]
                pltpu.async_copy(rows_v.at[b], dst, ssem.at[b], add=True)

                @pl.when(j + _NBUF < n_chunks)
                def _():
                    pltpu.make_async_copy(rows_v.at[b], dst,
                                          ssem.at[b]).wait()
                    fire(j + _NBUF, b)
            return 0

        lax.fori_loop(0, n_grp, body, 0)
        for b in range(_NBUF):
            j = (n_grp - 1) * _NBUF + b
            pltpu.make_async_copy(rows_v.at[b], acc.at[idx_v.at[j]],
                                  ssem.at[b]).wait()
        plsc.subcore_barrier()
        pltpu.sync_copy(acc.at[pl.ds(s * rows_per_tile, rows_per_tile)],
                        out_hbm.at[c, pl.ds(s * rows_per_tile, rows_per_tile)])

    return k(tp, src3, zeros_n)


def kernel(node_attr, edge_index, edge_attr, edge_sh, W1, b1, W2, b2,
           bn_weight, bn_bias):
    n_edges = edge_attr.shape[0]
    e_pad = _NW * _CHUNK * ((n_edges + _NW * _CHUNK - 1) // (_NW * _CHUNK))
    blk = 1280
    nblk = e_pad // blk
    r_blk = blk // 4

    na_pad = jnp.pad(node_attr, ((0, 0), (0, 4)))
    pad_e = e_pad - n_edges

    def pack_idx(idx):
        ip = jnp.pad(idx, (0, pad_e)).reshape(nblk, 4, r_blk)
        return ip.swapaxes(1, 2).reshape(_NW, -1, _CHUNK)

    dst3 = pack_idx(edge_index[1])
    src3 = pack_idx(edge_index[0])

    W2Ta = jnp.concatenate([W2.T, b2.reshape(400, 1)], axis=1)
    b1r = b1.reshape(1, 32)

    xg = _gather_call(na_pad, dst3, e_pad)
    xg_p = jnp.reshape(xg, (e_pad // 4, 128))
    tp_p = _edge_call(edge_attr, edge_sh, xg_p, W1, b1r, W2Ta,
                      n_edges, blk, e_pad)
    tp = jnp.reshape(tp_p, (e_pad, 32))
    acc2 = _scatter_call(tp, src3, jnp.zeros((N_NODES, 32), jnp.float32), e_pad)
    return _final_call(acc2, node_attr, bn_weight.reshape(1, 20),
                       bn_bias.reshape(1, 16))


# speedup vs baseline: 1.3011x; 1.3011x over previous
import functools
import math

import jax
import jax.numpy as jnp
from jax import lax
from jax.experimental import pallas as pl
from jax.experimental.pallas import tpu as pltpu
from jax.experimental.pallas import tpu_sc as plsc

N_NODES = 10000
MUL0 = 16
MUL1 = 4
D_IN = MUL0 + 3 * MUL1
W_N = 400

_INV_SQRT3 = 1.0 / math.sqrt(3.0)
_INV_SQRT20 = 1.0 / math.sqrt(float(MUL0 + MUL1))

_NW = 32
_CHUNK = 256


def _edge_body(n_edges, blk, ea_ref, sh_ref, xg_ref, w1t_ref, b1_ref,
               w2t_ref, o_ref):
    R = blk // 4
    eat = ea_ref[...]
    ht = jnp.maximum(
        jnp.dot(w1t_ref[...], eat, preferred_element_type=jnp.float32)
        + b1_ref[...], 0.0)
    ones = jnp.ones((1, blk), jnp.float32)
    hta = jnp.concatenate([ht, ones], axis=0).astype(jnp.bfloat16)
    wt = jnp.dot(w2t_ref[...], hta, preferred_element_type=jnp.float32)

    xp = xg_ref[...].T
    xgt = jnp.concatenate([xp[32 * k:32 * k + 32] for k in range(4)], axis=1)
    sht = sh_ref[...][0:4]
    ss = sht[0:1]
    xs = xgt[0:16]
    a_s = xs * ss
    dots = jnp.concatenate(
        [xgt[16 + 3 * i:17 + 3 * i] * sht[1:2]
         + xgt[17 + 3 * i:18 + 3 * i] * sht[2:3]
         + xgt[18 + 3 * i:19 + 3 * i] * sht[3:4] for i in range(4)],
        axis=0) * _INV_SQRT3

    out0 = a_s[0:1] * wt[0:16]
    for i in range(1, 16):
        out0 = out0 + a_s[i:i + 1] * wt[16 * i:16 * i + 16]
    for i in range(4):
        out0 = out0 + dots[i:i + 1] * wt[256 + 16 * i:256 + 16 * i + 16]
    out0 = out0 * _INV_SQRT20

    t3 = xs[0:1] * wt[320:324]
    for i in range(1, 16):
        t3 = t3 + xs[i:i + 1] * wt[320 + 4 * i:320 + 4 * i + 4]
    t4 = []
    for m in range(3):
        acc = xgt[16 + m:17 + m] * wt[384:388]
        for i in range(1, 4):
            acc = acc + xgt[16 + 3 * i + m:17 + 3 * i + m] \
                * wt[384 + 4 * i:384 + 4 * i + 4]
        t4.append(acc)

    base = pl.program_id(0) * blk
    eid = base + lax.broadcasted_iota(jnp.int32, (1, blk), 1)
    mask = jnp.where(eid < n_edges, 1.0, 0.0).astype(jnp.float32)
    rows = [out0]
    for j in range(4):
        for m in range(3):
            rows.append((t3[j:j + 1] * sht[1 + m:2 + m]
                         + t4[m][j:j + 1] * ss) * _INV_SQRT20)
    rows.append(mask)
    rows.append(jnp.zeros((3, blk), jnp.float32))
    ot = jnp.concatenate(rows, axis=0) * mask
    op = jnp.concatenate([ot[:, k * R:(k + 1) * R] for k in range(4)], axis=0)
    o_ref[...] = op.T


def _edge_call(eat, sht8, xg_p, W1T, b1c, W2Ta, n_edges, blk, e_pad,
               interpret=False):
    grid = e_pad // blk
    n_valid = n_edges // blk
    clamp = lambda i: (0, jnp.minimum(i, n_valid - 1))
    return pl.pallas_call(
        functools.partial(_edge_body, n_edges, blk),
        grid=(grid,),
        in_specs=[
            pl.BlockSpec((32, blk), clamp),
            pl.BlockSpec((8, blk), clamp),
            pl.BlockSpec((blk // 4, 128), lambda i: (i, 0)),
            pl.BlockSpec((32, 32), lambda i: (0, 0)),
            pl.BlockSpec((32, 1), lambda i: (0, 0)),
            pl.BlockSpec((400, 33), lambda i: (0, 0)),
        ],
        out_specs=pl.BlockSpec((blk // 4, 128), lambda i: (i, 0)),
        out_shape=jax.ShapeDtypeStruct((e_pad // 4, 128), jnp.float32),
        interpret=interpret,
    )(eat, sht8, xg_p, W1T, b1c, W2Ta)


def _final_body(acc_ref, na_ref, bnw_ref, bnb_ref, sel_ref, selt_ref, o_ref):
    acc = acc_ref[0] + acc_ref[1]
    cnt = jnp.maximum(acc[:, 28:29], 1.0)
    msg = acc[:, :28] / cnt
    out = msg + na_ref[...]
    s = out[:, :16]
    n = out.shape[0]
    mean = jnp.mean(s, axis=0, keepdims=True)
    var = jnp.mean(s * s, axis=0, keepdims=True) - mean * mean
    sn = (s - mean) * jax.lax.rsqrt(var + 1e-5) * bnw_ref[...][:, :16] \
        + bnb_ref[...]
    v = out[:, 16:28]
    fn = jnp.dot(jnp.sum(v * v, axis=0, keepdims=True), sel_ref[...],
                 preferred_element_type=jnp.float32) / (3.0 * n)
    scale4 = bnw_ref[...][:, 16:20] * jax.lax.rsqrt(fn + 1e-5)
    scale12 = jnp.dot(scale4, selt_ref[...],
                      preferred_element_type=jnp.float32)
    o_ref[...] = jnp.concatenate([sn, v * scale12], axis=1)


def _final_call(acc2, node_attr, bnw, bnb, interpret=False):
    sel = jnp.repeat(jnp.eye(4, dtype=jnp.float32), 3, axis=0)
    return pl.pallas_call(
        _final_body,
        out_shape=jax.ShapeDtypeStruct((N_NODES, 28), jnp.float32),
        interpret=interpret,
    )(acc2, node_attr, bnw, bnb, sel, sel.T)


_NBUF = 4


def _gather_call(nap_pad, dst3, e_pad, interpret=False):
    n_chunks = e_pad // (_NW * _CHUNK)
    n_grp = n_chunks // _NBUF
    mesh = plsc.VectorSubcoreMesh(core_axis_name="c", subcore_axis_name="s")

    @functools.partial(
        pl.kernel, mesh=mesh,
        out_type=jax.ShapeDtypeStruct((e_pad, 32), jnp.float32),
        scratch_types=[
            pltpu.VMEM((n_chunks, _CHUNK), jnp.int32),
            pltpu.VMEM((_NBUF, _CHUNK, 32), jnp.float32),
            pltpu.SemaphoreType.DMA((_NBUF,)),
            pltpu.SemaphoreType.DMA((_NBUF,)),
        ],
        compiler_params=pltpu.CompilerParams(use_tc_tiling_on_sc=False),
        interpret=interpret,
    )
    def k(nap_hbm, dst_hbm, out_hbm, idx_v, rows_v, gsem, osem):
        c = lax.axis_index("c")
        s = lax.axis_index("s")
        w = c * 16 + s
        pltpu.sync_copy(dst_hbm.at[w], idx_v)
        base = w * (n_chunks * _CHUNK)

        def fire(j, b):
            pltpu.async_copy(nap_hbm.at[idx_v.at[j]], rows_v.at[b],
                             gsem.at[b])

        for b in range(_NBUF):
            fire(b, b)

        def body(g, _):
            for b in range(_NBUF):
                j = g * _NBUF + b
                pltpu.make_async_copy(nap_hbm.at[idx_v.at[j]], rows_v.at[b],
                                      gsem.at[b]).wait()
                dst = out_hbm.at[pl.ds(base + j * _CHUNK, _CHUNK)]
                pltpu.async_copy(rows_v.at[b], dst, osem.at[b])

                @pl.when(j + _NBUF < n_chunks)
                def _():
                    pltpu.make_async_copy(rows_v.at[b], dst, osem.at[b]).wait()
                    fire(j + _NBUF, b)
            return 0

        lax.fori_loop(0, n_grp, body, 0)
        for b in range(_NBUF):
            j = (n_grp - 1) * _NBUF + b
            dst = out_hbm.at[pl.ds(base + j * _CHUNK, _CHUNK)]
            pltpu.make_async_copy(rows_v.at[b], dst, osem.at[b]).wait()

    return k(nap_pad, dst3)


def _scatter_call(tp, src3, zeros_n, e_pad, interpret=False):
    n_chunks = e_pad // (_NW * _CHUNK)
    rows_per_tile = N_NODES // 16
    mesh = plsc.VectorSubcoreMesh(core_axis_name="c", subcore_axis_name="s")

    @functools.partial(
        pl.kernel, mesh=mesh,
        out_type=jax.ShapeDtypeStruct((2, N_NODES, 32), jnp.float32),
        scratch_types=[
            pltpu.VMEM((n_chunks, _CHUNK), jnp.int32),
            pltpu.VMEM((_NBUF, _CHUNK, 32), jnp.float32),
            pltpu.VMEM_SHARED((N_NODES, 32), jnp.float32),
            pltpu.SemaphoreType.DMA((_NBUF,)),
            pltpu.SemaphoreType.DMA((_NBUF,)),
        ],
        compiler_params=pltpu.CompilerParams(use_tc_tiling_on_sc=False),
        interpret=interpret,
    )
    def k(tp_hbm, src_hbm, zeros_hbm, out_hbm, idx_v, rows_v, acc, gsem, ssem):
        c = lax.axis_index("c")
        s = lax.axis_index("s")
        w = c * 16 + s
        pltpu.sync_copy(zeros_hbm.at[pl.ds(s * rows_per_tile, rows_per_tile)],
                        acc.at[pl.ds(s * rows_per_tile, rows_per_tile)])
        plsc.subcore_barrier()
        pltpu.sync_copy(src_hbm.at[w], idx_v)
        base = w * (n_chunks * _CHUNK)
        n_grp = n_chunks // _NBUF

        def fire(j, b):
            pltpu.async_copy(tp_hbm.at[pl.ds(base + j * _CHUNK, _CHUNK)],
                             rows_v.at[b], gsem.at[b])

        for b in range(_NBUF):
            fire(b, b)

        def body(g, _):
            for b in range(_NBUF):
                j = g * _NBUF + b
                pltpu.make_async_copy(
                    tp_hbm.at[pl.ds(base + j * _CHUNK, _CHUNK)],
                    rows_v.at[b], gsem.at[b]).wait()
                dst = acc.at[idx_v.at[j]]
                pltpu.async_copy(rows_v.at[b], dst, ssem.at[b], add=True)

                @pl.when(j + _NBUF < n_chunks)
                def _():
                    pltpu.make_async_copy(rows_v.at[b], dst,
                                          ssem.at[b]).wait()
                    fire(j + _NBUF, b)
            return 0

        lax.fori_loop(0, n_grp, body, 0)
        for b in range(_NBUF):
            j = (n_grp - 1) * _NBUF + b
            pltpu.make_async_copy(rows_v.at[b], acc.at[idx_v.at[j]],
                                  ssem.at[b]).wait()
        plsc.subcore_barrier()
        pltpu.sync_copy(acc.at[pl.ds(s * rows_per_tile, rows_per_tile)],
                        out_hbm.at[c, pl.ds(s * rows_per_tile, rows_per_tile)])

    return k(tp, src3, zeros_n)


def kernel(node_attr, edge_index, edge_attr, edge_sh, W1, b1, W2, b2,
           bn_weight, bn_bias):
    n_edges = edge_attr.shape[0]
    e_pad = _NW * _CHUNK * ((n_edges + _NW * _CHUNK - 1) // (_NW * _CHUNK))
    blk = 1280
    nblk = e_pad // blk
    r_blk = blk // 4

    na_pad = jnp.pad(node_attr, ((0, 0), (0, 4)))
    pad_e = e_pad - n_edges

    def pack_idx(idx):
        ip = jnp.pad(idx, (0, pad_e)).reshape(nblk, 4, r_blk)
        return ip.swapaxes(1, 2).reshape(_NW, -1, _CHUNK)

    dst3 = pack_idx(edge_index[1])
    src3 = pack_idx(edge_index[0])

    W2Ta = jnp.concatenate([W2.T, b2.reshape(400, 1)],
                           axis=1).astype(jnp.bfloat16)
    eat = edge_attr.T
    sht8 = jnp.pad(edge_sh, ((0, 0), (0, 4))).T

    xg = _gather_call(na_pad, dst3, e_pad)
    xg_p = jnp.reshape(xg, (e_pad // 4, 128))
    tp_p = _edge_call(eat, sht8, xg_p, W1.T, b1.reshape(32, 1), W2Ta,
                      n_edges, blk, e_pad)
    tp = jnp.reshape(tp_p, (e_pad, 32))
    acc2 = _scatter_call(tp, src3, jnp.zeros((N_NODES, 32), jnp.float32), e_pad)
    return _final_call(acc2, node_attr, bn_weight.reshape(1, 20),
                       bn_bias.reshape(1, 16))
